# Initial kernel scaffold; baseline (speedup 1.0000x reference)
import jax
import jax.numpy as jnp
from jax.experimental import pallas as pl


def kernel(feats, adjs, W1, b1, W2, b2, W_ih, W_hh, b_ih, b_hh, Wlin, blin):
    raise NotImplementedError("write your pallas kernel here")


# speedup vs baseline: 3.2757x; 3.2757x over previous
import functools

import jax
import jax.numpy as jnp
from jax import lax
from jax.experimental import pallas as pl
from jax.experimental.pallas import tpu as pltpu
from jax.experimental.pallas import tpu_sc as plsc

N = 10000
E = 320000
T = 4
D_IN = 128
D_H = 128
D_OUT = 64

NPAD = 10240
CHUNK = 128
CPT = 80
EPAD = 32 * CPT * CHUNK
ROWS_PT = NPAD // 16
GARBAGE = N

_mesh = plsc.VectorSubcoreMesh(core_axis_name="c", subcore_axis_name="s")


@functools.partial(
    pl.kernel,
    out_type=jax.ShapeDtypeStruct((2, T, NPAD, 16), jnp.float32),
    mesh=_mesh,
    scratch_types=[
        pltpu.VMEM((CHUNK, 16), jnp.float32),
        pltpu.VMEM((NPAD // 16, 16), jnp.float32),
        pltpu.VMEM((CPT, CHUNK), jnp.int32),
        pltpu.VMEM_SHARED((NPAD, 16), jnp.float32),
    ],
)
def _sc_degree(dst_hbm, ones_hbm, zeros_hbm, out_hbm, ones, zstage, idxb,
               acc):
  c = lax.axis_index("c")
  s = lax.axis_index("s")
  wid = c * 16 + s
  rows_pt = NPAD // 16
  pltpu.sync_copy(ones_hbm, ones)
  pltpu.sync_copy(zeros_hbm, zstage)

  for t in range(T):
    pltpu.sync_copy(zstage, acc.at[pl.ds(s * rows_pt, rows_pt)])
    pltpu.sync_copy(dst_hbm.at[wid * T + t], idxb)
    plsc.subcore_barrier()
    for j in range(CPT):
      pltpu.sync_copy(ones, acc.at[idxb.at[j]], add=True)
    plsc.subcore_barrier()

    @pl.when(s == 0)
    def _():
      pltpu.sync_copy(acc, out_hbm.at[c, t])

    plsc.subcore_barrier()


@functools.partial(
    pl.kernel,
    out_type=jax.ShapeDtypeStruct((2, NPAD, D_H), jnp.float32),
    mesh=_mesh,
    scratch_types=[
        pltpu.VMEM((CHUNK, D_H), jnp.float32),
        pltpu.VMEM((64, D_H), jnp.float32),
        pltpu.VMEM((CPT // 2, CHUNK), jnp.int32),
        pltpu.VMEM((CPT // 2, CHUNK), jnp.int32),
        pltpu.VMEM_SHARED((NPAD, D_H), jnp.float32),
    ],
)
def _sc_propagate(y_hbm, src_hbm, dst_hbm, zeros_hbm, out_hbm, rows, zstage,
                  idxs, idxd, acc):
  c = lax.axis_index("c")
  s = lax.axis_index("s")
  wid = c * 16 + s
  pltpu.sync_copy(zeros_hbm, zstage)

  @pl.loop(0, ROWS_PT, step=64)
  def _(r):
    pltpu.sync_copy(zstage, acc.at[pl.ds(s * ROWS_PT + r, 64)])

  plsc.subcore_barrier()

  for h in range(2):
    pltpu.sync_copy(src_hbm.at[wid * 2 + h], idxs)
    pltpu.sync_copy(dst_hbm.at[wid * 2 + h], idxd)
    for j in range(CPT // 2):
      pltpu.sync_copy(y_hbm.at[idxs.at[j]], rows)
      pltpu.sync_copy(rows, acc.at[idxd.at[j]], add=True)

  plsc.subcore_barrier()

  @pl.when(s == 0)
  def _():
    pltpu.sync_copy(acc, out_hbm.at[c])


def _dinv_of(dp):
  deg = dp[0, :, 0:1] + dp[1, :, 0:1] + 1.0
  return lax.rsqrt(deg)


BN = 2000


def _prep_body(f_ref, w_ref, dp_ref, z_ref):
  dinv = _dinv_of(dp_ref[0])
  z_ref[0] = jnp.dot(f_ref[0], w_ref[...],
                     preferred_element_type=jnp.float32) * dinv


def _tc_prep(feats, W1T, degp):
  return pl.pallas_call(
      _prep_body,
      grid=(T, N // BN),
      in_specs=[
          pl.BlockSpec((1, BN, D_IN), lambda t, i: (t, i, 0)),
          pl.BlockSpec((D_IN, D_H), lambda t, i: (0, 0)),
          pl.BlockSpec((1, 2, BN, 16), lambda t, i: (t, 0, i, 0)),
      ],
      out_specs=pl.BlockSpec((1, BN, D_H), lambda t, i: (t, i, 0)),
      out_shape=jax.ShapeDtypeStruct((T, N, D_H), jnp.float32),
  )(feats, W1T, degp)


def _layer2_body(s_ref, z_ref, dp_ref, w_ref, b_ref, o_ref):
  dinv = _dinv_of(dp_ref)
  x1 = jax.nn.relu(dinv * (s_ref[0] + s_ref[1] + z_ref[...]) + b_ref[...])
  o_ref[...] = jnp.dot(x1, w_ref[...],
                       preferred_element_type=jnp.float32) * dinv


def _tc_layer2(s1, z1t, degpt, W2T, b1r):
  return pl.pallas_call(
      _layer2_body,
      grid=(N // BN,),
      in_specs=[
          pl.BlockSpec((2, BN, D_H), lambda i: (0, i, 0)),
          pl.BlockSpec((BN, D_H), lambda i: (i, 0)),
          pl.BlockSpec((2, BN, 16), lambda i: (0, i, 0)),
          pl.BlockSpec((D_H, D_H), lambda i: (0, 0)),
          pl.BlockSpec((1, D_H), lambda i: (0, 0)),
      ],
      out_specs=pl.BlockSpec((BN, D_H), lambda i: (i, 0)),
      out_shape=jax.ShapeDtypeStruct((N, D_H), jnp.float32),
  )(s1, z1t, degpt, W2T, b1r)


def _lstm_body(z0, z1, z2, z3, s0, s1, s2, s3, dp_ref, b2r, wih, whh, bihr,
               bhhr, wlin, blinr, o_ref):
  zs = (z0, z1, z2, z3)
  ss = (s0, s1, s2, s3)
  bn = z0.shape[0]
  h = jnp.zeros((bn, D_H), jnp.float32)
  c = jnp.zeros((bn, D_H), jnp.float32)
  bias = bihr[...] + bhhr[...]
  for t in range(T):
    dinv = _dinv_of(dp_ref[t])
    emb = jax.nn.relu(dinv * (ss[t][0] + ss[t][1] + zs[t][...]) + b2r[...])
    g = (jnp.dot(emb, wih[...], preferred_element_type=jnp.float32) +
         jnp.dot(h, whh[...], preferred_element_type=jnp.float32) + bias)
    i_ = jax.nn.sigmoid(g[:, 0:D_H])
    f_ = jax.nn.sigmoid(g[:, D_H:2 * D_H])
    g_ = jnp.tanh(g[:, 2 * D_H:3 * D_H])
    o_ = jax.nn.sigmoid(g[:, 3 * D_H:4 * D_H])
    c = f_ * c + i_ * g_
    h = o_ * jnp.tanh(c)
  logits = jnp.dot(h, wlin[...], preferred_element_type=jnp.float32) + blinr[...]
  m = jnp.max(logits, axis=1, keepdims=True)
  lse = jnp.log(jnp.sum(jnp.exp(logits - m), axis=1, keepdims=True)) + m
  o_ref[...] = logits - lse


def _tc_lstm(z2s, s2s, degp, b2r, WihT, WhhT, bihr, bhhr, WlinT, blinr):
  zspec = pl.BlockSpec((BN, D_H), lambda i: (i, 0))
  sspec = pl.BlockSpec((2, BN, D_H), lambda i: (0, i, 0))
  full = lambda shape: pl.BlockSpec(shape, lambda i: tuple(0 for _ in shape))
  return pl.pallas_call(
      _lstm_body,
      grid=(N // BN,),
      in_specs=([zspec] * T + [sspec] * T + [
          pl.BlockSpec((T, 2, BN, 16), lambda i: (0, 0, i, 0)),
          full((1, D_H)),
          full((D_H, 4 * D_H)),
          full((D_H, 4 * D_H)),
          full((1, 4 * D_H)),
          full((1, 4 * D_H)),
          full((D_H, D_OUT)),
          full((1, D_OUT)),
      ]),
      out_specs=pl.BlockSpec((BN, D_OUT), lambda i: (i, 0)),
      out_shape=jax.ShapeDtypeStruct((N, D_OUT), jnp.float32),
  )(*z2s, *s2s, degp, b2r, WihT, WhhT, bihr, bhhr, WlinT, blinr)


def _xla_count(dst_t):
  return jnp.zeros((N,), jnp.float32).at[dst_t].add(1.0)


def _xla_propagate(y, src_t, dst_t):
  s = jnp.zeros((N, D_H), jnp.float32).at[dst_t].add(y[src_t])
  return jnp.zeros((2, NPAD, D_H), jnp.float32).at[0, :N].set(s)


def kernel(feats, adjs, W1, b1, W2, b2, W_ih, W_hh, b_ih, b_hh, Wlin, blin):
  src = adjs[:, 0, :]
  dst = adjs[:, 1, :]

  cnt = jax.vmap(_xla_count)(dst)
  degp = jnp.zeros((T, 2, NPAD, 16), jnp.float32).at[:, 0, :N, 0].set(cnt)

  z1 = _tc_prep(feats, W1.T, degp)

  b1r = b1.reshape(1, D_H)
  z2s, s2s = [], []
  for t in range(T):
    s1 = _xla_propagate(z1[t], src[t], dst[t])
    z2 = _tc_layer2(s1, z1[t], degp[t], W2.T, b1r)
    s2 = _xla_propagate(z2, src[t], dst[t])
    z2s.append(z2)
    s2s.append(s2)

  return _tc_lstm(z2s, s2s, degp, b2.reshape(1, D_H), W_ih.T, W_hh.T,
                  b_ih.reshape(1, 4 * D_H), b_hh.reshape(1, 4 * D_H),
                  Wlin.T, blin.reshape(1, D_OUT))
